# baseline (speedup 1.0000x reference)
import functools

import numpy as np
import jax
import jax.numpy as jnp
from jax import lax
from jax.experimental import pallas as pl
from jax.experimental.pallas import tpu as pltpu
from jax.experimental.pallas import tpu_sc as plsc

_D_MODEL = 256
_N_HEADS = 8
_D_HEAD = _D_MODEL // _N_HEADS
_NUM_POINTS_LIST = (4, 4, 4)
_OFFSET_SCALE = 0.5
_SPATIAL_SHAPES = ((80, 80), (40, 40), (20, 20))
_B = 16
_Q = 300
_S = sum(h * w for h, w in _SPATIAL_SHAPES)
_SUM_POINTS = sum(_NUM_POINTS_LIST)
_HP = _N_HEADS * _SUM_POINTS
_BQ = _B * _Q
_BHALF = _B // 2
_BQH = _BHALF * _Q
_N_ROWSH = _BHALF * _S * _N_HEADS

_np_cols = np.arange(_HP)
_p_of_col = _np_cols % _SUM_POINTS
_h_of_col = _np_cols // _SUM_POINTS
_lvl_starts = np.cumsum([0] + [n for n in _NUM_POINTS_LIST])[:-1]
_lvl_of_p = np.searchsorted(_lvl_starts, _p_of_col, side="right") - 1
_W_of_col = np.array([_SPATIAL_SHAPES[l][1] for l in _lvl_of_p], np.float32)
_H_of_col = np.array([_SPATIAL_SHAPES[l][0] for l in _lvl_of_p], np.float32)
_s0_sizes = np.cumsum([0] + [h * w for h, w in _SPATIAL_SHAPES])[:-1]
_s0_of_col = np.array([_s0_sizes[l] for l in _lvl_of_p], np.float32)
_nps_of_col = np.array(
    [1.0 / _NUM_POINTS_LIST[l] for l in _lvl_of_p], np.float32)
_CONST5 = np.stack([
    _nps_of_col,
    _W_of_col,
    _H_of_col,
    _s0_of_col,
    _h_of_col.astype(np.float32),
]).astype(np.float32)
_PA = np.zeros((_D_MODEL, 128), np.float32)
_PB = np.zeros((_D_MODEL, 128), np.float32)
for _j in range(128):
    _PA[(_j // 16) * 32 + _j % 16, _j] = 1.0
    _PB[(_j // 16) * 32 + 16 + _j % 16, _j] = 1.0
_SEG = (( _np_cols[:, None] // _SUM_POINTS)
        == (_np_cols[None, :] // _SUM_POINTS)).astype(np.float32)
_PERM = np.zeros((_HP, _N_HEADS * 16), np.float32)
for _j in range(_HP):
    _PERM[_j, (_j // _SUM_POINTS) * 16 + (_j % _SUM_POINTS)] = 1.0


def _prep_body(hid_ref, rp_ref, wx_ref, wy_ref, wa_ref, seg_ref, perm_ref,
               cv_ref, attn_ref, idx_ref, wgt_ref):
    b = pl.program_id(0)
    hs = hid_ref[0]
    offx = jnp.dot(hs, wx_ref[...], preferred_element_type=jnp.float32)
    offy = jnp.dot(hs, wy_ref[...], preferred_element_type=jnp.float32)
    cv = cv_ref[...]
    nps = cv[0:1, :]
    wl = cv[1:2, :]
    hl = cv[2:3, :]
    s0l = cv[3:4, :]
    hcol = cv[4:5, :]
    offx = offx + cv[5:6, :]
    offy = offy + cv[6:7, :]
    logits = jnp.dot(hs, wa_ref[...], preferred_element_type=jnp.float32)
    logits = logits + cv[7:8, :]
    m = jnp.max(logits, axis=1, keepdims=True)
    e = jnp.exp(logits - m)
    ssum = jnp.dot(e, seg_ref[...], preferred_element_type=jnp.float32)
    attn = e / ssum
    attn_ref[0] = attn

    rp = rp_ref[0]
    refx = rp[:, 0:1]
    refy = rp[:, 1:2]
    refw = rp[:, 2:3]
    refh = rp[:, 3:4]
    locx = refx + ((offx * nps) * refw) * _OFFSET_SCALE
    locy = refy + ((offy * nps) * refh) * _OFFSET_SCALE
    gx = 2.0 * locx - 1.0
    gy = 2.0 * locy - 1.0
    x = (gx + 1.0) * wl / 2.0 - 0.5
    y = (gy + 1.0) * hl / 2.0 - 0.5
    x0 = jnp.floor(x)
    y0 = jnp.floor(y)
    wx1 = x - x0
    wx0 = 1.0 - wx1
    wy1 = y - y0
    wy0 = 1.0 - wy1
    base = (b * _S).astype(jnp.float32)
    for c, (dx, dy, wxc, wyc) in enumerate(
            [(0.0, 0.0, wx0, wy0), (1.0, 0.0, wx1, wy0),
             (0.0, 1.0, wx0, wy1), (1.0, 1.0, wx1, wy1)]):
        xc = x0 + dx
        yc = y0 + dy
        mask = ((xc >= 0.0) & (xc <= wl - 1.0)
                & (yc >= 0.0) & (yc <= hl - 1.0))
        xi = jnp.clip(xc, 0.0, wl - 1.0)
        yi = jnp.clip(yc, 0.0, hl - 1.0)
        row = (base + (s0l + yi * wl + xi)) * float(_N_HEADS) + hcol
        idx_ref[0, :, c, :] = row.astype(jnp.int32)
        w96 = attn * wxc * wyc * jnp.where(mask, 1.0, 0.0)
        wgt_ref[0, :, c, :] = jnp.dot(
            w96, perm_ref[...], preferred_element_type=jnp.float32)


def _make_prep(interpret=False):
    return pl.pallas_call(
        _prep_body,
        grid=(_B,),
        in_specs=[
            pl.BlockSpec((1, _Q, _D_MODEL), lambda b: (b, 0, 0)),
            pl.BlockSpec((1, _Q, 4), lambda b: (b, 0, 0)),
            pl.BlockSpec((_D_MODEL, _HP), lambda b: (0, 0)),
            pl.BlockSpec((_D_MODEL, _HP), lambda b: (0, 0)),
            pl.BlockSpec((_D_MODEL, _HP), lambda b: (0, 0)),
            pl.BlockSpec((_HP, _HP), lambda b: (0, 0)),
            pl.BlockSpec((_HP, _N_HEADS * 16), lambda b: (0, 0)),
            pl.BlockSpec((8, _HP), lambda b: (0, 0)),
        ],
        out_specs=[
            pl.BlockSpec((1, _Q, _HP), lambda b: (b, 0, 0)),
            pl.BlockSpec((1, _Q, 4, _HP), lambda b: (b, 0, 0, 0)),
            pl.BlockSpec((1, _Q, 4, _N_HEADS * 16), lambda b: (b, 0, 0, 0)),
        ],
        out_shape=[
            jax.ShapeDtypeStruct((_B, _Q, _HP), jnp.float32),
            jax.ShapeDtypeStruct((_B, _Q, 4, _HP), jnp.int32),
            jax.ShapeDtypeStruct((_B, _Q, 4, _N_HEADS * 16), jnp.float32),
        ],
        interpret=interpret,
    )


_RL_G = 4480
_RL_STEPS = _B * _S // _RL_G


def _pack_body(in_ref, pa_ref, pb_ref, out_ref):
    xb = in_ref[...].astype(jnp.bfloat16)
    a = jnp.dot(xb, pa_ref[...], preferred_element_type=jnp.float32)
    b = jnp.dot(xb, pb_ref[...], preferred_element_type=jnp.float32)
    au = jax.lax.bitcast_convert_type(a, jnp.uint32)
    bu = jax.lax.bitcast_convert_type(b, jnp.uint32)
    w = jnp.bitwise_or(jnp.right_shift(au, 16),
                       jnp.bitwise_and(bu, jnp.uint32(0xFFFF0000)))
    out_ref[...] = jax.lax.bitcast_convert_type(w, jnp.int32)


def _make_pack(interpret=False):
    return pl.pallas_call(
        _pack_body,
        grid=(_RL_STEPS,),
        in_specs=[
            pl.BlockSpec((_RL_G, _D_MODEL), lambda b: (b, 0)),
            pl.BlockSpec((_D_MODEL, 128), lambda b: (0, 0)),
            pl.BlockSpec((_D_MODEL, 128), lambda b: (0, 0)),
        ],
        out_specs=pl.BlockSpec((_RL_G, 128), lambda b: (b, 0)),
        out_shape=jax.ShapeDtypeStruct((_B * _S, 128), jnp.int32),
        interpret=interpret,
    )


_NW = 32
_GPW = _BQ // _NW
_NB = 5
_NBLK = _GPW // _NB


def _sc_body(table, idx_hbm, wgt_hbm, out_hbm, idx_v, wgt_v, rows_v, out_v,
             sem_r0, sem_r1, sem_i):
    wid = lax.axis_index("s") * 2 + lax.axis_index("c")
    g0 = wid * _GPW
    sem_r = (sem_r0, sem_r1)

    def issue_rows(buf, t):
        for q in range(_NB):
            for c in range(4):
                pltpu.async_copy(table.at[idx_v.at[buf, q, c]],
                                 rows_v.at[buf, q, c], sem_r[buf])

    def wait_rows(buf):
        for q in range(_NB):
            for c in range(4):
                pltpu.make_async_copy(table.at[pl.ds(0, _HP)],
                                      rows_v.at[buf, q, c],
                                      sem_r[buf]).wait()

    def issue_idxw(buf, t):
        base = g0 + t * _NB
        pltpu.async_copy(idx_hbm.at[pl.ds(base, _NB)], idx_v.at[buf], sem_i)
        pltpu.async_copy(wgt_hbm.at[pl.ds(base, _NB)], wgt_v.at[buf], sem_i)

    def wait_idxw(buf):
        pltpu.make_async_copy(idx_hbm.at[pl.ds(0, _NB)], idx_v.at[buf],
                              sem_i).wait()
        pltpu.make_async_copy(wgt_hbm.at[pl.ds(0, _NB)], wgt_v.at[buf],
                              sem_i).wait()

    def compute(buf, t):
        def qbody(q, carry):
            def hbody(hh, carry2):
                h0 = hh * 4
                for dh in range(4):
                    h = h0 + dh
                    acc0 = jnp.zeros((16,), jnp.float32)
                    acc1 = jnp.zeros((16,), jnp.float32)
                    for c in range(4):
                        wv = wgt_v[buf, q, c, h, :]
                        for p in range(_SUM_POINTS):
                            r = h * _SUM_POINTS + p
                            w = wv[p]
                            v = rows_v[buf, q, c, r, 0:16]
                            lo = jax.lax.bitcast_convert_type(
                                jnp.left_shift(v, 16), jnp.float32)
                            hi = jax.lax.bitcast_convert_type(
                                jnp.bitwise_and(v, jnp.int32(-65536)),
                                jnp.float32)
                            acc0 = acc0 + w * lo
                            acc1 = acc1 + w * hi
                    out_v[q, pl.ds(h * _D_HEAD, 16)] = acc0
                    out_v[q, pl.ds(h * _D_HEAD + 16, 16)] = acc1
                return carry2

            lax.fori_loop(0, 2, hbody, 0)
            return carry

        lax.fori_loop(0, _NB, qbody, 0)
        pltpu.sync_copy(out_v, out_hbm.at[pl.ds(g0 + t * _NB, _NB)])

    pltpu.sync_copy(idx_hbm.at[pl.ds(g0, _NB)], idx_v.at[0])
    pltpu.sync_copy(wgt_hbm.at[pl.ds(g0, _NB)], wgt_v.at[0])
    issue_rows(0, 0)
    issue_idxw(1, 1)

    def pair(t2, carry):
        for par in (0, 1):
            t = 2 * t2 + par
            nxt = 1 - par

            @pl.when(t < _NBLK - 1)
            def _():
                wait_idxw(nxt)
                issue_rows(nxt, t + 1)

            wait_rows(par)
            compute(par, t)

            @pl.when(t < _NBLK - 2)
            def _():
                issue_idxw(par, t + 2)

        return carry

    lax.fori_loop(0, _NBLK // 2, pair, 0)
    if _NBLK % 2:
        wait_rows(0)
        compute(0, _NBLK - 1)


@functools.cache
def _make_sc_gather():
    return functools.partial(
        pl.kernel,
        out_type=jax.ShapeDtypeStruct((_BQ, _D_MODEL), jnp.float32),
        mesh=plsc.VectorSubcoreMesh(core_axis_name="c", subcore_axis_name="s"),
        compiler_params=pltpu.CompilerParams(use_tc_tiling_on_sc=False),
        scratch_types=[
            pltpu.VMEM((2, _NB, 4, _HP), jnp.int32),
            pltpu.VMEM((2, _NB, 4, _N_HEADS, 16), jnp.float32),
            pltpu.VMEM((2, _NB, 4, _HP, 16), jnp.int32),
            pltpu.VMEM((_NB, _D_MODEL), jnp.float32),
            pltpu.SemaphoreType.DMA,
            pltpu.SemaphoreType.DMA,
            pltpu.SemaphoreType.DMA,
        ],
    )(_sc_body)


def kernel(hidden_states, encoder_hidden_states, reference_points, W_off,
           b_off, W_attn, b_attn, spatial_shapes):
    del spatial_shapes
    rp = reference_points.reshape(_B, _Q, 4)
    woffx = W_off[:, 0::2]
    woffy = W_off[:, 1::2]
    cv = jnp.concatenate([
        jnp.asarray(_CONST5),
        b_off[0::2][None, :],
        b_off[1::2][None, :],
        b_attn[None, :],
    ], axis=0)
    attn96, idx, wgt = _make_prep()(
        hidden_states, rp, woffx, woffy, W_attn, jnp.asarray(_SEG),
        jnp.asarray(_PERM), cv)
    table = _make_pack()(
        encoder_hidden_states.reshape(_B * _S, _D_MODEL),
        jnp.asarray(_PA, jnp.bfloat16), jnp.asarray(_PB, jnp.bfloat16))
    table = table.reshape(_B * _S * _N_HEADS, 16)
    out = _make_sc_gather()(
        table, idx.reshape(_BQ, 4, _HP),
        wgt.reshape(_BQ, 4, _N_HEADS, 16))
    return (out.reshape(_B, _Q, _D_MODEL),
            attn96.reshape(_B, _Q, _N_HEADS, _SUM_POINTS))


# speedup vs baseline: 1.0201x; 1.0201x over previous
import functools

import numpy as np
import jax
import jax.numpy as jnp
from jax import lax
from jax.experimental import pallas as pl
from jax.experimental.pallas import tpu as pltpu
from jax.experimental.pallas import tpu_sc as plsc

_D_MODEL = 256
_N_HEADS = 8
_D_HEAD = _D_MODEL // _N_HEADS
_NUM_POINTS_LIST = (4, 4, 4)
_OFFSET_SCALE = 0.5
_SPATIAL_SHAPES = ((80, 80), (40, 40), (20, 20))
_B = 16
_Q = 300
_S = sum(h * w for h, w in _SPATIAL_SHAPES)
_SUM_POINTS = sum(_NUM_POINTS_LIST)
_HP = _N_HEADS * _SUM_POINTS
_BQ = _B * _Q
_BHALF = _B // 2
_BQH = _BHALF * _Q
_N_ROWSH = _BHALF * _S * _N_HEADS

_np_cols = np.arange(_HP)
_p_of_col = _np_cols % _SUM_POINTS
_h_of_col = _np_cols // _SUM_POINTS
_lvl_starts = np.cumsum([0] + [n for n in _NUM_POINTS_LIST])[:-1]
_lvl_of_p = np.searchsorted(_lvl_starts, _p_of_col, side="right") - 1
_W_of_col = np.array([_SPATIAL_SHAPES[l][1] for l in _lvl_of_p], np.float32)
_H_of_col = np.array([_SPATIAL_SHAPES[l][0] for l in _lvl_of_p], np.float32)
_s0_sizes = np.cumsum([0] + [h * w for h, w in _SPATIAL_SHAPES])[:-1]
_s0_of_col = np.array([_s0_sizes[l] for l in _lvl_of_p], np.float32)
_nps_of_col = np.array(
    [1.0 / _NUM_POINTS_LIST[l] for l in _lvl_of_p], np.float32)
_CONST5 = np.stack([
    _nps_of_col,
    _W_of_col,
    _H_of_col,
    _s0_of_col,
    _h_of_col.astype(np.float32),
]).astype(np.float32)
_PA = np.zeros((_D_MODEL, 128), np.float32)
_PB = np.zeros((_D_MODEL, 128), np.float32)
for _j in range(128):
    _PA[(_j // 16) * 32 + _j % 16, _j] = 1.0
    _PB[(_j // 16) * 32 + 16 + _j % 16, _j] = 1.0
_SEG = (( _np_cols[:, None] // _SUM_POINTS)
        == (_np_cols[None, :] // _SUM_POINTS)).astype(np.float32)
_PERM = np.zeros((_HP, _N_HEADS * 16), np.float32)
for _j in range(_HP):
    _PERM[_j, (_j // _SUM_POINTS) * 16 + (_j % _SUM_POINTS)] = 1.0


def _prep_body(hid_ref, rp_ref, wx_ref, wy_ref, wa_ref, seg_ref, perm_ref,
               cv_ref, attn_ref, idx_ref, wgt_ref):
    b = pl.program_id(0)
    hs = hid_ref[0]
    offx = jnp.dot(hs, wx_ref[...], preferred_element_type=jnp.float32)
    offy = jnp.dot(hs, wy_ref[...], preferred_element_type=jnp.float32)
    cv = cv_ref[...]
    nps = cv[0:1, :]
    wl = cv[1:2, :]
    hl = cv[2:3, :]
    s0l = cv[3:4, :]
    hcol = cv[4:5, :]
    offx = offx + cv[5:6, :]
    offy = offy + cv[6:7, :]
    logits = jnp.dot(hs, wa_ref[...], preferred_element_type=jnp.float32)
    logits = logits + cv[7:8, :]
    m = jnp.max(logits, axis=1, keepdims=True)
    e = jnp.exp(logits - m)
    ssum = jnp.dot(e, seg_ref[...], preferred_element_type=jnp.float32)
    attn = e / ssum
    attn_ref[0] = attn

    rp = rp_ref[0]
    refx = rp[:, 0:1]
    refy = rp[:, 1:2]
    refw = rp[:, 2:3]
    refh = rp[:, 3:4]
    locx = refx + ((offx * nps) * refw) * _OFFSET_SCALE
    locy = refy + ((offy * nps) * refh) * _OFFSET_SCALE
    gx = 2.0 * locx - 1.0
    gy = 2.0 * locy - 1.0
    x = (gx + 1.0) * wl / 2.0 - 0.5
    y = (gy + 1.0) * hl / 2.0 - 0.5
    x0 = jnp.floor(x)
    y0 = jnp.floor(y)
    wx1 = x - x0
    wx0 = 1.0 - wx1
    wy1 = y - y0
    wy0 = 1.0 - wy1
    base = (b * _S).astype(jnp.float32)
    for c, (dx, dy, wxc, wyc) in enumerate(
            [(0.0, 0.0, wx0, wy0), (1.0, 0.0, wx1, wy0),
             (0.0, 1.0, wx0, wy1), (1.0, 1.0, wx1, wy1)]):
        xc = x0 + dx
        yc = y0 + dy
        mask = ((xc >= 0.0) & (xc <= wl - 1.0)
                & (yc >= 0.0) & (yc <= hl - 1.0))
        xi = jnp.clip(xc, 0.0, wl - 1.0)
        yi = jnp.clip(yc, 0.0, hl - 1.0)
        row = (base + (s0l + yi * wl + xi)) * float(_N_HEADS) + hcol
        idx_ref[0, :, c, :] = row.astype(jnp.int32)
        w96 = attn * wxc * wyc * jnp.where(mask, 1.0, 0.0)
        wgt_ref[0, :, c, :] = jnp.dot(
            w96, perm_ref[...], preferred_element_type=jnp.float32)


def _make_prep(interpret=False):
    return pl.pallas_call(
        _prep_body,
        grid=(_B,),
        in_specs=[
            pl.BlockSpec((1, _Q, _D_MODEL), lambda b: (b, 0, 0)),
            pl.BlockSpec((1, _Q, 4), lambda b: (b, 0, 0)),
            pl.BlockSpec((_D_MODEL, _HP), lambda b: (0, 0)),
            pl.BlockSpec((_D_MODEL, _HP), lambda b: (0, 0)),
            pl.BlockSpec((_D_MODEL, _HP), lambda b: (0, 0)),
            pl.BlockSpec((_HP, _HP), lambda b: (0, 0)),
            pl.BlockSpec((_HP, _N_HEADS * 16), lambda b: (0, 0)),
            pl.BlockSpec((8, _HP), lambda b: (0, 0)),
        ],
        out_specs=[
            pl.BlockSpec((1, _Q, _HP), lambda b: (b, 0, 0)),
            pl.BlockSpec((1, _Q, 4, _HP), lambda b: (b, 0, 0, 0)),
            pl.BlockSpec((1, _Q, 4, _N_HEADS * 16), lambda b: (b, 0, 0, 0)),
        ],
        out_shape=[
            jax.ShapeDtypeStruct((_B, _Q, _HP), jnp.float32),
            jax.ShapeDtypeStruct((_B, _Q, 4, _HP), jnp.int32),
            jax.ShapeDtypeStruct((_B, _Q, 4, _N_HEADS * 16), jnp.float32),
        ],
        interpret=interpret,
    )


_RL_G = 13440
_RL_STEPS = _B * _S // _RL_G


def _pack_body(in_ref, pa_ref, pb_ref, out_ref):
    xb = in_ref[...].astype(jnp.bfloat16)
    a = jnp.dot(xb, pa_ref[...], preferred_element_type=jnp.float32)
    b = jnp.dot(xb, pb_ref[...], preferred_element_type=jnp.float32)
    au = jax.lax.bitcast_convert_type(a, jnp.uint32)
    bu = jax.lax.bitcast_convert_type(b, jnp.uint32)
    w = jnp.bitwise_or(jnp.right_shift(au, 16),
                       jnp.bitwise_and(bu, jnp.uint32(0xFFFF0000)))
    out_ref[...] = jax.lax.bitcast_convert_type(w, jnp.int32)


def _make_pack(interpret=False):
    return pl.pallas_call(
        _pack_body,
        grid=(_RL_STEPS,),
        in_specs=[
            pl.BlockSpec((_RL_G, _D_MODEL), lambda b: (b, 0)),
            pl.BlockSpec((_D_MODEL, 128), lambda b: (0, 0)),
            pl.BlockSpec((_D_MODEL, 128), lambda b: (0, 0)),
        ],
        out_specs=pl.BlockSpec((_RL_G, 128), lambda b: (b, 0)),
        out_shape=jax.ShapeDtypeStruct((_B * _S, 128), jnp.int32),
        interpret=interpret,
    )


_NW = 32
_GPW = _BQ // _NW
_NB = 5
_NBLK = _GPW // _NB


def _sc_body(table, idx_hbm, wgt_hbm, out_hbm, idx_v, wgt_v, rows_v, out_v,
             sem_r0, sem_r1, sem_i):
    wid = lax.axis_index("s") * 2 + lax.axis_index("c")
    g0 = wid * _GPW
    sem_r = (sem_r0, sem_r1)

    def issue_rows(buf, t):
        for q in range(_NB):
            for c in range(4):
                pltpu.async_copy(table.at[idx_v.at[buf, q, c]],
                                 rows_v.at[buf, q, c], sem_r[buf])

    def wait_rows(buf):
        for q in range(_NB):
            for c in range(4):
                pltpu.make_async_copy(table.at[pl.ds(0, _HP)],
                                      rows_v.at[buf, q, c],
                                      sem_r[buf]).wait()

    def issue_idxw(buf, t):
        base = g0 + t * _NB
        pltpu.async_copy(idx_hbm.at[pl.ds(base, _NB)], idx_v.at[buf], sem_i)
        pltpu.async_copy(wgt_hbm.at[pl.ds(base, _NB)], wgt_v.at[buf], sem_i)

    def wait_idxw(buf):
        pltpu.make_async_copy(idx_hbm.at[pl.ds(0, _NB)], idx_v.at[buf],
                              sem_i).wait()
        pltpu.make_async_copy(wgt_hbm.at[pl.ds(0, _NB)], wgt_v.at[buf],
                              sem_i).wait()

    def compute(buf, t):
        def qbody(q, carry):
            def hbody(hh, carry2):
                h0 = hh * 4
                for dh in range(4):
                    h = h0 + dh
                    acc0 = jnp.zeros((16,), jnp.float32)
                    acc1 = jnp.zeros((16,), jnp.float32)
                    for c in range(4):
                        wv = wgt_v[buf, q, c, h, :]
                        for p in range(_SUM_POINTS):
                            r = h * _SUM_POINTS + p
                            w = wv[p]
                            v = rows_v[buf, q, c, r, 0:16]
                            lo = jax.lax.bitcast_convert_type(
                                jnp.left_shift(v, 16), jnp.float32)
                            hi = jax.lax.bitcast_convert_type(
                                jnp.bitwise_and(v, jnp.int32(-65536)),
                                jnp.float32)
                            acc0 = acc0 + w * lo
                            acc1 = acc1 + w * hi
                    out_v[q, pl.ds(h * _D_HEAD, 16)] = acc0
                    out_v[q, pl.ds(h * _D_HEAD + 16, 16)] = acc1
                return carry2

            lax.fori_loop(0, 2, hbody, 0)
            return carry

        lax.fori_loop(0, _NB, qbody, 0)
        pltpu.sync_copy(out_v, out_hbm.at[pl.ds(g0 + t * _NB, _NB)])

    pltpu.sync_copy(idx_hbm.at[pl.ds(g0, _NB)], idx_v.at[0])
    pltpu.sync_copy(wgt_hbm.at[pl.ds(g0, _NB)], wgt_v.at[0])
    issue_rows(0, 0)
    issue_idxw(1, 1)

    def pair(t2, carry):
        for par in (0, 1):
            t = 2 * t2 + par
            nxt = 1 - par

            @pl.when(t < _NBLK - 1)
            def _():
                wait_idxw(nxt)
                issue_rows(nxt, t + 1)

            wait_rows(par)
            compute(par, t)

            @pl.when(t < _NBLK - 2)
            def _():
                issue_idxw(par, t + 2)

        return carry

    lax.fori_loop(0, _NBLK // 2, pair, 0)
    if _NBLK % 2:
        wait_rows(0)
        compute(0, _NBLK - 1)


@functools.cache
def _make_sc_gather():
    return functools.partial(
        pl.kernel,
        out_type=jax.ShapeDtypeStruct((_BQ, _D_MODEL), jnp.float32),
        mesh=plsc.VectorSubcoreMesh(core_axis_name="c", subcore_axis_name="s"),
        compiler_params=pltpu.CompilerParams(use_tc_tiling_on_sc=False),
        scratch_types=[
            pltpu.VMEM((2, _NB, 4, _HP), jnp.int32),
            pltpu.VMEM((2, _NB, 4, _N_HEADS, 16), jnp.float32),
            pltpu.VMEM((2, _NB, 4, _HP, 16), jnp.int32),
            pltpu.VMEM((_NB, _D_MODEL), jnp.float32),
            pltpu.SemaphoreType.DMA,
            pltpu.SemaphoreType.DMA,
            pltpu.SemaphoreType.DMA,
        ],
    )(_sc_body)


def kernel(hidden_states, encoder_hidden_states, reference_points, W_off,
           b_off, W_attn, b_attn, spatial_shapes):
    del spatial_shapes
    rp = reference_points.reshape(_B, _Q, 4)
    woffx = W_off[:, 0::2]
    woffy = W_off[:, 1::2]
    cv = jnp.concatenate([
        jnp.asarray(_CONST5),
        b_off[0::2][None, :],
        b_off[1::2][None, :],
        b_attn[None, :],
    ], axis=0)
    attn96, idx, wgt = _make_prep()(
        hidden_states, rp, woffx, woffy, W_attn, jnp.asarray(_SEG),
        jnp.asarray(_PERM), cv)
    table = _make_pack()(
        encoder_hidden_states.reshape(_B * _S, _D_MODEL),
        jnp.asarray(_PA, jnp.bfloat16), jnp.asarray(_PB, jnp.bfloat16))
    table = table.reshape(_B * _S * _N_HEADS, 16)
    out = _make_sc_gather()(
        table, idx.reshape(_BQ, 4, _HP),
        wgt.reshape(_BQ, 4, _N_HEADS, 16))
    return (out.reshape(_B, _Q, _D_MODEL),
            attn96.reshape(_B, _Q, _N_HEADS, _SUM_POINTS))


# speedup vs baseline: 1.0280x; 1.0077x over previous
import functools

import numpy as np
import jax
import jax.numpy as jnp
from jax import lax
from jax.experimental import pallas as pl
from jax.experimental.pallas import tpu as pltpu
from jax.experimental.pallas import tpu_sc as plsc

_D_MODEL = 256
_N_HEADS = 8
_D_HEAD = _D_MODEL // _N_HEADS
_NUM_POINTS_LIST = (4, 4, 4)
_OFFSET_SCALE = 0.5
_SPATIAL_SHAPES = ((80, 80), (40, 40), (20, 20))
_B = 16
_Q = 300
_S = sum(h * w for h, w in _SPATIAL_SHAPES)
_SUM_POINTS = sum(_NUM_POINTS_LIST)
_HP = _N_HEADS * _SUM_POINTS
_BQ = _B * _Q
_BHALF = _B // 2
_BQH = _BHALF * _Q
_N_ROWSH = _BHALF * _S * _N_HEADS

_np_cols = np.arange(_HP)
_p_of_col = _np_cols % _SUM_POINTS
_h_of_col = _np_cols // _SUM_POINTS
_lvl_starts = np.cumsum([0] + [n for n in _NUM_POINTS_LIST])[:-1]
_lvl_of_p = np.searchsorted(_lvl_starts, _p_of_col, side="right") - 1
_W_of_col = np.array([_SPATIAL_SHAPES[l][1] for l in _lvl_of_p], np.float32)
_H_of_col = np.array([_SPATIAL_SHAPES[l][0] for l in _lvl_of_p], np.float32)
_s0_sizes = np.cumsum([0] + [h * w for h, w in _SPATIAL_SHAPES])[:-1]
_s0_of_col = np.array([_s0_sizes[l] for l in _lvl_of_p], np.float32)
_nps_of_col = np.array(
    [1.0 / _NUM_POINTS_LIST[l] for l in _lvl_of_p], np.float32)
_CONST5 = np.stack([
    _nps_of_col,
    _W_of_col,
    _H_of_col,
    _s0_of_col,
    _h_of_col.astype(np.float32),
]).astype(np.float32)
_PA = np.zeros((_D_MODEL, 128), np.float32)
_PB = np.zeros((_D_MODEL, 128), np.float32)
for _j in range(128):
    _PA[(_j // 16) * 32 + _j % 16, _j] = 1.0
    _PB[(_j // 16) * 32 + 16 + _j % 16, _j] = 1.0
_SEG = (( _np_cols[:, None] // _SUM_POINTS)
        == (_np_cols[None, :] // _SUM_POINTS)).astype(np.float32)
_PERM = np.zeros((_HP, _N_HEADS * 16), np.float32)
for _j in range(_HP):
    _PERM[_j, (_j // _SUM_POINTS) * 16 + (_j % _SUM_POINTS)] = 1.0


def _prep_body(hid_ref, rp_ref, wx_ref, wy_ref, wa_ref, seg_ref, perm_ref,
               cv_ref, attn_ref, idx_ref, wgt_ref):
    b = pl.program_id(0)
    hs = hid_ref[0]
    offx = jnp.dot(hs, wx_ref[...], preferred_element_type=jnp.float32)
    offy = jnp.dot(hs, wy_ref[...], preferred_element_type=jnp.float32)
    cv = cv_ref[...]
    nps = cv[0:1, :]
    wl = cv[1:2, :]
    hl = cv[2:3, :]
    s0l = cv[3:4, :]
    hcol = cv[4:5, :]
    offx = offx + cv[5:6, :]
    offy = offy + cv[6:7, :]
    logits = jnp.dot(hs, wa_ref[...], preferred_element_type=jnp.float32)
    logits = logits + cv[7:8, :]
    m = jnp.max(logits, axis=1, keepdims=True)
    e = jnp.exp(logits - m)
    ssum = jnp.dot(e, seg_ref[...], preferred_element_type=jnp.float32)
    attn = e / ssum
    attn_ref[0] = attn

    rp = rp_ref[0]
    refx = rp[:, 0:1]
    refy = rp[:, 1:2]
    refw = rp[:, 2:3]
    refh = rp[:, 3:4]
    locx = refx + ((offx * nps) * refw) * _OFFSET_SCALE
    locy = refy + ((offy * nps) * refh) * _OFFSET_SCALE
    gx = 2.0 * locx - 1.0
    gy = 2.0 * locy - 1.0
    x = (gx + 1.0) * wl / 2.0 - 0.5
    y = (gy + 1.0) * hl / 2.0 - 0.5
    x0 = jnp.floor(x)
    y0 = jnp.floor(y)
    wx1 = x - x0
    wx0 = 1.0 - wx1
    wy1 = y - y0
    wy0 = 1.0 - wy1
    base = (b * _S).astype(jnp.float32)
    for c, (dx, dy, wxc, wyc) in enumerate(
            [(0.0, 0.0, wx0, wy0), (1.0, 0.0, wx1, wy0),
             (0.0, 1.0, wx0, wy1), (1.0, 1.0, wx1, wy1)]):
        xc = x0 + dx
        yc = y0 + dy
        mask = ((xc >= 0.0) & (xc <= wl - 1.0)
                & (yc >= 0.0) & (yc <= hl - 1.0))
        xi = jnp.clip(xc, 0.0, wl - 1.0)
        yi = jnp.clip(yc, 0.0, hl - 1.0)
        row = (base + (s0l + yi * wl + xi)) * float(_N_HEADS) + hcol
        idx_ref[0, :, c, :] = row.astype(jnp.int32)
        w96 = attn * wxc * wyc * jnp.where(mask, 1.0, 0.0)
        wgt_ref[0, :, c, :] = jnp.dot(
            w96, perm_ref[...], preferred_element_type=jnp.float32)


def _make_prep(interpret=False):
    return pl.pallas_call(
        _prep_body,
        grid=(_B,),
        in_specs=[
            pl.BlockSpec((1, _Q, _D_MODEL), lambda b: (b, 0, 0)),
            pl.BlockSpec((1, _Q, 4), lambda b: (b, 0, 0)),
            pl.BlockSpec((_D_MODEL, _HP), lambda b: (0, 0)),
            pl.BlockSpec((_D_MODEL, _HP), lambda b: (0, 0)),
            pl.BlockSpec((_D_MODEL, _HP), lambda b: (0, 0)),
            pl.BlockSpec((_HP, _HP), lambda b: (0, 0)),
            pl.BlockSpec((_HP, _N_HEADS * 16), lambda b: (0, 0)),
            pl.BlockSpec((8, _HP), lambda b: (0, 0)),
        ],
        out_specs=[
            pl.BlockSpec((1, _Q, _HP), lambda b: (b, 0, 0)),
            pl.BlockSpec((1, _Q, 4, _HP), lambda b: (b, 0, 0, 0)),
            pl.BlockSpec((1, _Q, 4, _N_HEADS * 16), lambda b: (b, 0, 0, 0)),
        ],
        out_shape=[
            jax.ShapeDtypeStruct((_B, _Q, _HP), jnp.float32),
            jax.ShapeDtypeStruct((_B, _Q, 4, _HP), jnp.int32),
            jax.ShapeDtypeStruct((_B, _Q, 4, _N_HEADS * 16), jnp.float32),
        ],
        interpret=interpret,
    )


_RL_G = 13440
_RL_STEPS = _B * _S // _RL_G


def _pack_body(in_ref, pa_ref, pb_ref, out_ref):
    xb = in_ref[...].astype(jnp.bfloat16)
    a = jnp.dot(xb, pa_ref[...], preferred_element_type=jnp.float32)
    b = jnp.dot(xb, pb_ref[...], preferred_element_type=jnp.float32)
    au = jax.lax.bitcast_convert_type(a, jnp.uint32)
    bu = jax.lax.bitcast_convert_type(b, jnp.uint32)
    w = jnp.bitwise_or(jnp.right_shift(au, 16),
                       jnp.bitwise_and(bu, jnp.uint32(0xFFFF0000)))
    out_ref[...] = jax.lax.bitcast_convert_type(w, jnp.int32)


def _make_pack(interpret=False):
    return pl.pallas_call(
        _pack_body,
        grid=(_RL_STEPS,),
        in_specs=[
            pl.BlockSpec((_RL_G, _D_MODEL), lambda b: (b, 0)),
            pl.BlockSpec((_D_MODEL, 128), lambda b: (0, 0)),
            pl.BlockSpec((_D_MODEL, 128), lambda b: (0, 0)),
        ],
        out_specs=pl.BlockSpec((_RL_G, 128), lambda b: (b, 0)),
        out_shape=jax.ShapeDtypeStruct((_B * _S, 128), jnp.int32),
        interpret=interpret,
    )


_NW = 32
_GPW = _BQ // _NW
_NB = 6
_NBLK = _GPW // _NB


def _sc_body(table, idx_hbm, wgt_hbm, out_hbm, idx_v, wgt_v, rows_v, out_v,
             sem_r0, sem_r1, sem_i):
    wid = lax.axis_index("s") * 2 + lax.axis_index("c")
    g0 = wid * _GPW
    sem_r = (sem_r0, sem_r1)

    def issue_rows(buf, t):
        for q in range(_NB):
            for c in range(4):
                pltpu.async_copy(table.at[idx_v.at[buf, q, c]],
                                 rows_v.at[buf, q, c], sem_r[buf])

    def wait_rows(buf):
        for q in range(_NB):
            for c in range(4):
                pltpu.make_async_copy(table.at[pl.ds(0, _HP)],
                                      rows_v.at[buf, q, c],
                                      sem_r[buf]).wait()

    def issue_idxw(buf, t):
        base = g0 + t * _NB
        pltpu.async_copy(idx_hbm.at[pl.ds(base, _NB)], idx_v.at[buf], sem_i)
        pltpu.async_copy(wgt_hbm.at[pl.ds(base, _NB)], wgt_v.at[buf], sem_i)

    def wait_idxw(buf):
        pltpu.make_async_copy(idx_hbm.at[pl.ds(0, _NB)], idx_v.at[buf],
                              sem_i).wait()
        pltpu.make_async_copy(wgt_hbm.at[pl.ds(0, _NB)], wgt_v.at[buf],
                              sem_i).wait()

    def compute(buf, t):
        def qbody(q, carry):
            def hbody(hh, carry2):
                h0 = hh * 4
                for dh in range(4):
                    h = h0 + dh
                    acc0 = jnp.zeros((16,), jnp.float32)
                    acc1 = jnp.zeros((16,), jnp.float32)
                    for c in range(4):
                        wv = wgt_v[buf, q, c, h, :]
                        for p in range(_SUM_POINTS):
                            r = h * _SUM_POINTS + p
                            w = wv[p]
                            v = rows_v[buf, q, c, r, 0:16]
                            lo = jax.lax.bitcast_convert_type(
                                jnp.left_shift(v, 16), jnp.float32)
                            hi = jax.lax.bitcast_convert_type(
                                jnp.bitwise_and(v, jnp.int32(-65536)),
                                jnp.float32)
                            acc0 = acc0 + w * lo
                            acc1 = acc1 + w * hi
                    out_v[q, pl.ds(h * _D_HEAD, 16)] = acc0
                    out_v[q, pl.ds(h * _D_HEAD + 16, 16)] = acc1
                return carry2

            lax.fori_loop(0, 2, hbody, 0)
            return carry

        lax.fori_loop(0, _NB, qbody, 0)
        pltpu.sync_copy(out_v, out_hbm.at[pl.ds(g0 + t * _NB, _NB)])

    pltpu.sync_copy(idx_hbm.at[pl.ds(g0, _NB)], idx_v.at[0])
    pltpu.sync_copy(wgt_hbm.at[pl.ds(g0, _NB)], wgt_v.at[0])
    issue_rows(0, 0)
    issue_idxw(1, 1)

    def pair(t2, carry):
        for par in (0, 1):
            t = 2 * t2 + par
            nxt = 1 - par

            @pl.when(t < _NBLK - 1)
            def _():
                wait_idxw(nxt)
                issue_rows(nxt, t + 1)

            wait_rows(par)
            compute(par, t)

            @pl.when(t < _NBLK - 2)
            def _():
                issue_idxw(par, t + 2)

        return carry

    lax.fori_loop(0, _NBLK // 2, pair, 0)
    if _NBLK % 2:
        wait_rows(0)
        compute(0, _NBLK - 1)


@functools.cache
def _make_sc_gather():
    return functools.partial(
        pl.kernel,
        out_type=jax.ShapeDtypeStruct((_BQ, _D_MODEL), jnp.float32),
        mesh=plsc.VectorSubcoreMesh(core_axis_name="c", subcore_axis_name="s"),
        compiler_params=pltpu.CompilerParams(use_tc_tiling_on_sc=False),
        scratch_types=[
            pltpu.VMEM((2, _NB, 4, _HP), jnp.int32),
            pltpu.VMEM((2, _NB, 4, _N_HEADS, 16), jnp.float32),
            pltpu.VMEM((2, _NB, 4, _HP, 16), jnp.int32),
            pltpu.VMEM((_NB, _D_MODEL), jnp.float32),
            pltpu.SemaphoreType.DMA,
            pltpu.SemaphoreType.DMA,
            pltpu.SemaphoreType.DMA,
        ],
    )(_sc_body)


def kernel(hidden_states, encoder_hidden_states, reference_points, W_off,
           b_off, W_attn, b_attn, spatial_shapes):
    del spatial_shapes
    rp = reference_points.reshape(_B, _Q, 4)
    woffx = W_off[:, 0::2]
    woffy = W_off[:, 1::2]
    cv = jnp.concatenate([
        jnp.asarray(_CONST5),
        b_off[0::2][None, :],
        b_off[1::2][None, :],
        b_attn[None, :],
    ], axis=0)
    attn96, idx, wgt = _make_prep()(
        hidden_states, rp, woffx, woffy, W_attn, jnp.asarray(_SEG),
        jnp.asarray(_PERM), cv)
    table = _make_pack()(
        encoder_hidden_states.reshape(_B * _S, _D_MODEL),
        jnp.asarray(_PA, jnp.bfloat16), jnp.asarray(_PB, jnp.bfloat16))
    table = table.reshape(_B * _S * _N_HEADS, 16)
    out = _make_sc_gather()(
        table, idx.reshape(_BQ, 4, _HP),
        wgt.reshape(_BQ, 4, _N_HEADS, 16))
    return (out.reshape(_B, _Q, _D_MODEL),
            attn96.reshape(_B, _Q, _N_HEADS, _SUM_POINTS))


# speedup vs baseline: 1.0286x; 1.0006x over previous
import functools

import numpy as np
import jax
import jax.numpy as jnp
from jax import lax
from jax.experimental import pallas as pl
from jax.experimental.pallas import tpu as pltpu
from jax.experimental.pallas import tpu_sc as plsc

_D_MODEL = 256
_N_HEADS = 8
_D_HEAD = _D_MODEL // _N_HEADS
_NUM_POINTS_LIST = (4, 4, 4)
_OFFSET_SCALE = 0.5
_SPATIAL_SHAPES = ((80, 80), (40, 40), (20, 20))
_B = 16
_Q = 300
_S = sum(h * w for h, w in _SPATIAL_SHAPES)
_SUM_POINTS = sum(_NUM_POINTS_LIST)
_HP = _N_HEADS * _SUM_POINTS
_BQ = _B * _Q

_np_cols = np.arange(_HP)
_p_of_col = _np_cols % _SUM_POINTS
_h_of_col = _np_cols // _SUM_POINTS
_lvl_starts = np.cumsum([0] + [n for n in _NUM_POINTS_LIST])[:-1]
_lvl_of_p = np.searchsorted(_lvl_starts, _p_of_col, side="right") - 1
_W_of_col = np.array([_SPATIAL_SHAPES[l][1] for l in _lvl_of_p], np.float32)
_H_of_col = np.array([_SPATIAL_SHAPES[l][0] for l in _lvl_of_p], np.float32)
_s0_sizes = np.cumsum([0] + [h * w for h, w in _SPATIAL_SHAPES])[:-1]
_s0_of_col = np.array([_s0_sizes[l] for l in _lvl_of_p], np.float32)
_nps_of_col = np.array(
    [1.0 / _NUM_POINTS_LIST[l] for l in _lvl_of_p], np.float32)
_CONST5 = np.stack([
    _nps_of_col,
    _W_of_col,
    _H_of_col,
    _s0_of_col,
    _h_of_col.astype(np.float32),
]).astype(np.float32)
_PA = np.zeros((_D_MODEL, 128), np.float32)
_PB = np.zeros((_D_MODEL, 128), np.float32)
for _j in range(128):
    _PA[(_j // 16) * 32 + _j % 16, _j] = 1.0
    _PB[(_j // 16) * 32 + 16 + _j % 16, _j] = 1.0
_SEG = (( _np_cols[:, None] // _SUM_POINTS)
        == (_np_cols[None, :] // _SUM_POINTS)).astype(np.float32)
_PERM = np.zeros((_HP, _N_HEADS * 16), np.float32)
for _j in range(_HP):
    _PERM[_j, (_j // _SUM_POINTS) * 16 + (_j % _SUM_POINTS)] = 1.0


def _prep_body(hid_ref, rp_ref, wx_ref, wy_ref, wa_ref, seg_ref, perm_ref,
               cv_ref, attn_ref, idx_ref, wgt_ref):
    b = pl.program_id(0)
    hs = hid_ref[0]
    offx = jnp.dot(hs, wx_ref[...], preferred_element_type=jnp.float32)
    offy = jnp.dot(hs, wy_ref[...], preferred_element_type=jnp.float32)
    cv = cv_ref[...]
    nps = cv[0:1, :]
    wl = cv[1:2, :]
    hl = cv[2:3, :]
    s0l = cv[3:4, :]
    hcol = cv[4:5, :]
    offx = offx + cv[5:6, :]
    offy = offy + cv[6:7, :]
    logits = jnp.dot(hs, wa_ref[...], preferred_element_type=jnp.float32)
    logits = logits + cv[7:8, :]
    m = jnp.max(logits, axis=1, keepdims=True)
    e = jnp.exp(logits - m)
    ssum = jnp.dot(e, seg_ref[...], preferred_element_type=jnp.float32)
    attn = e / ssum
    attn_ref[0] = attn

    rp = rp_ref[0]
    refx = rp[:, 0:1]
    refy = rp[:, 1:2]
    refw = rp[:, 2:3]
    refh = rp[:, 3:4]
    locx = refx + ((offx * nps) * refw) * _OFFSET_SCALE
    locy = refy + ((offy * nps) * refh) * _OFFSET_SCALE
    gx = 2.0 * locx - 1.0
    gy = 2.0 * locy - 1.0
    x = (gx + 1.0) * wl / 2.0 - 0.5
    y = (gy + 1.0) * hl / 2.0 - 0.5
    x0 = jnp.floor(x)
    y0 = jnp.floor(y)
    wx1 = x - x0
    wx0 = 1.0 - wx1
    wy1 = y - y0
    wy0 = 1.0 - wy1
    base = (b * _S).astype(jnp.float32)
    for c, (dx, dy, wxc, wyc) in enumerate(
            [(0.0, 0.0, wx0, wy0), (1.0, 0.0, wx1, wy0),
             (0.0, 1.0, wx0, wy1), (1.0, 1.0, wx1, wy1)]):
        xc = x0 + dx
        yc = y0 + dy
        mask = ((xc >= 0.0) & (xc <= wl - 1.0)
                & (yc >= 0.0) & (yc <= hl - 1.0))
        xi = jnp.clip(xc, 0.0, wl - 1.0)
        yi = jnp.clip(yc, 0.0, hl - 1.0)
        row = (base + (s0l + yi * wl + xi)) * float(_N_HEADS) + hcol
        idx_ref[0, :, c, :] = row.astype(jnp.int32)
        w96 = attn * wxc * wyc * jnp.where(mask, 1.0, 0.0)
        wgt_ref[0, :, c, :] = jnp.dot(
            w96, perm_ref[...], preferred_element_type=jnp.float32)


def _make_prep(interpret=False):
    return pl.pallas_call(
        _prep_body,
        grid=(_B,),
        in_specs=[
            pl.BlockSpec((1, _Q, _D_MODEL), lambda b: (b, 0, 0)),
            pl.BlockSpec((1, _Q, 4), lambda b: (b, 0, 0)),
            pl.BlockSpec((_D_MODEL, _HP), lambda b: (0, 0)),
            pl.BlockSpec((_D_MODEL, _HP), lambda b: (0, 0)),
            pl.BlockSpec((_D_MODEL, _HP), lambda b: (0, 0)),
            pl.BlockSpec((_HP, _HP), lambda b: (0, 0)),
            pl.BlockSpec((_HP, _N_HEADS * 16), lambda b: (0, 0)),
            pl.BlockSpec((8, _HP), lambda b: (0, 0)),
        ],
        out_specs=[
            pl.BlockSpec((1, _Q, _HP), lambda b: (b, 0, 0)),
            pl.BlockSpec((1, _Q, 4, _HP), lambda b: (b, 0, 0, 0)),
            pl.BlockSpec((1, _Q, 4, _N_HEADS * 16), lambda b: (b, 0, 0, 0)),
        ],
        out_shape=[
            jax.ShapeDtypeStruct((_B, _Q, _HP), jnp.float32),
            jax.ShapeDtypeStruct((_B, _Q, 4, _HP), jnp.int32),
            jax.ShapeDtypeStruct((_B, _Q, 4, _N_HEADS * 16), jnp.float32),
        ],
        interpret=interpret,
    )


_RL_G = 13440
_RL_STEPS = _B * _S // _RL_G


def _pack_body(in_ref, pa_ref, pb_ref, out_ref):
    xb = in_ref[...].astype(jnp.bfloat16)
    a = jnp.dot(xb, pa_ref[...], preferred_element_type=jnp.float32)
    b = jnp.dot(xb, pb_ref[...], preferred_element_type=jnp.float32)
    au = jax.lax.bitcast_convert_type(a, jnp.uint32)
    bu = jax.lax.bitcast_convert_type(b, jnp.uint32)
    w = jnp.bitwise_or(jnp.right_shift(au, 16),
                       jnp.bitwise_and(bu, jnp.uint32(0xFFFF0000)))
    out_ref[...] = jax.lax.bitcast_convert_type(w, jnp.int32)


def _make_pack(interpret=False):
    return pl.pallas_call(
        _pack_body,
        grid=(_RL_STEPS,),
        in_specs=[
            pl.BlockSpec((_RL_G, _D_MODEL), lambda b: (b, 0)),
            pl.BlockSpec((_D_MODEL, 128), lambda b: (0, 0)),
            pl.BlockSpec((_D_MODEL, 128), lambda b: (0, 0)),
        ],
        out_specs=pl.BlockSpec((_RL_G, 128), lambda b: (b, 0)),
        out_shape=jax.ShapeDtypeStruct((_B * _S, 128), jnp.int32),
        interpret=interpret,
    )


_NW = 32
_GPW = _BQ // _NW
_NB = 6
_NBLK = _GPW // _NB


def _sc_body(table, idx_hbm, wgt_hbm, out_hbm, idx_v, wgt_v, rows_v, out_v,
             sem_r0, sem_r1, sem_i):
    wid = lax.axis_index("s") * 2 + lax.axis_index("c")
    g0 = wid * _GPW
    sem_r = (sem_r0, sem_r1)

    def issue_rows(buf, t):
        for q in range(_NB):
            for c in range(4):
                pltpu.async_copy(table.at[idx_v.at[buf, q, c]],
                                 rows_v.at[buf, q, c], sem_r[buf])

    def wait_rows(buf):
        for q in range(_NB):
            for c in range(4):
                pltpu.make_async_copy(table.at[pl.ds(0, _HP)],
                                      rows_v.at[buf, q, c],
                                      sem_r[buf]).wait()

    def issue_idxw(buf, t):
        base = g0 + t * _NB
        pltpu.async_copy(idx_hbm.at[pl.ds(base, _NB)], idx_v.at[buf], sem_i)
        pltpu.async_copy(wgt_hbm.at[pl.ds(base, _NB)], wgt_v.at[buf], sem_i)

    def wait_idxw(buf):
        pltpu.make_async_copy(idx_hbm.at[pl.ds(0, _NB)], idx_v.at[buf],
                              sem_i).wait()
        pltpu.make_async_copy(wgt_hbm.at[pl.ds(0, _NB)], wgt_v.at[buf],
                              sem_i).wait()

    def compute(buf, t):
        def qbody(q, carry):
            def hbody(hh, carry2):
                h0 = hh * 4
                for dh in range(4):
                    h = h0 + dh
                    acc0 = jnp.zeros((16,), jnp.float32)
                    acc1 = jnp.zeros((16,), jnp.float32)
                    for c in range(4):
                        wv = wgt_v[buf, q, c, h, :]
                        for p in range(_SUM_POINTS):
                            r = h * _SUM_POINTS + p
                            w = wv[p]
                            v = rows_v[buf, q, c, r, 0:16]
                            lo = jax.lax.bitcast_convert_type(
                                jnp.left_shift(v, 16), jnp.float32)
                            hi = jax.lax.bitcast_convert_type(
                                jnp.bitwise_and(v, jnp.int32(-65536)),
                                jnp.float32)
                            acc0 = acc0 + w * lo
                            acc1 = acc1 + w * hi
                    out_v[q, pl.ds(h * _D_HEAD, 16)] = acc0
                    out_v[q, pl.ds(h * _D_HEAD + 16, 16)] = acc1
                return carry2

            lax.fori_loop(0, 2, hbody, 0)
            return carry

        lax.fori_loop(0, _NB, qbody, 0)
        pltpu.sync_copy(out_v, out_hbm.at[pl.ds(g0 + t * _NB, _NB)])

    pltpu.sync_copy(idx_hbm.at[pl.ds(g0, _NB)], idx_v.at[0])
    pltpu.sync_copy(wgt_hbm.at[pl.ds(g0, _NB)], wgt_v.at[0])
    issue_rows(0, 0)
    issue_idxw(1, 1)

    def pair(t2, carry):
        for par in (0, 1):
            t = 2 * t2 + par
            nxt = 1 - par

            @pl.when(t < _NBLK - 1)
            def _():
                wait_idxw(nxt)
                issue_rows(nxt, t + 1)

            wait_rows(par)
            compute(par, t)

            @pl.when(t < _NBLK - 2)
            def _():
                issue_idxw(par, t + 2)

        return carry

    lax.fori_loop(0, _NBLK // 2, pair, 0)
    if _NBLK % 2:
        wait_rows(0)
        compute(0, _NBLK - 1)


@functools.cache
def _make_sc_gather():
    return functools.partial(
        pl.kernel,
        out_type=jax.ShapeDtypeStruct((_BQ, _D_MODEL), jnp.float32),
        mesh=plsc.VectorSubcoreMesh(core_axis_name="c", subcore_axis_name="s"),
        compiler_params=pltpu.CompilerParams(use_tc_tiling_on_sc=False),
        scratch_types=[
            pltpu.VMEM((2, _NB, 4, _HP), jnp.int32),
            pltpu.VMEM((2, _NB, 4, _N_HEADS, 16), jnp.float32),
            pltpu.VMEM((2, _NB, 4, _HP, 16), jnp.int32),
            pltpu.VMEM((_NB, _D_MODEL), jnp.float32),
            pltpu.SemaphoreType.DMA,
            pltpu.SemaphoreType.DMA,
            pltpu.SemaphoreType.DMA,
        ],
    )(_sc_body)


def kernel(hidden_states, encoder_hidden_states, reference_points, W_off,
           b_off, W_attn, b_attn, spatial_shapes):
    del spatial_shapes
    rp = reference_points.reshape(_B, _Q, 4)
    woffx = W_off[:, 0::2]
    woffy = W_off[:, 1::2]
    cv = jnp.concatenate([
        jnp.asarray(_CONST5),
        b_off[0::2][None, :],
        b_off[1::2][None, :],
        b_attn[None, :],
    ], axis=0)
    attn96, idx, wgt = _make_prep()(
        hidden_states, rp, woffx, woffy, W_attn, jnp.asarray(_SEG),
        jnp.asarray(_PERM), cv)
    table = _make_pack()(
        encoder_hidden_states.reshape(_B * _S, _D_MODEL),
        jnp.asarray(_PA, jnp.bfloat16), jnp.asarray(_PB, jnp.bfloat16))
    table = table.reshape(_B * _S * _N_HEADS, 16)
    out = _make_sc_gather()(
        table, idx.reshape(_BQ, 4, _HP),
        wgt.reshape(_BQ, 4, _N_HEADS, 16))
    return (out.reshape(_B, _Q, _D_MODEL),
            attn96.reshape(_B, _Q, _N_HEADS, _SUM_POINTS))
